# baseline (speedup 1.0000x reference)
import functools

import jax
import jax.numpy as jnp
from jax import lax
from jax.experimental import pallas as pl
from jax.experimental.pallas import tpu as pltpu
from jax.experimental.pallas import tpu_sc as plsc

N_NODES = 10000
E_EDGES = 640000
G_GRAPHS = 64

NC, NS, LANES = 2, 16, 16
NW = NC * NS
CH = 96
K = 3
CPW0 = 144
CPW1 = 276
E_PAD = NS * (CPW0 + CPW1) * CH
F = 128

STRIPE = 624
LAST0 = NS * STRIPE
REM = N_NODES - LAST0

BROWS = 1000
NBLK = N_NODES // BROWS


@functools.lru_cache(maxsize=None)
def _make_segsum(FR):
    mesh = plsc.VectorSubcoreMesh(
        core_axis_name="c", subcore_axis_name="s", num_cores=NC, num_subcores=NS
    )

    @functools.partial(
        pl.kernel,
        out_type=jax.ShapeDtypeStruct((NC, N_NODES, F), jnp.float32),
        mesh=mesh,
        scratch_types=(
            [pltpu.VMEM_SHARED((N_NODES, F), jnp.float32)]
            + [pltpu.VMEM((4 * K, CH), jnp.int32)]
            + [pltpu.VMEM((2 * K, CH), jnp.float32)]
            + [pltpu.VMEM((CH, F), jnp.float32) for _ in range(K)]
            + [pltpu.SemaphoreType.DMA] * (4 * K)
        ),
        compiler_params=pltpu.CompilerParams(needs_layout_passes=False),
    )
    def segsum(y_hbm, src_hbm, dst_hbm, w_hbm, out_hbm, acc, eall, wall,
               *bufs):
        g = list(bufs[0:K])
        isem = list(bufs[K:3 * K])
        gsem = list(bufs[3 * K:4 * K])
        ssem = list(bufs[4 * K:5 * K])
        c = lax.axis_index("c")
        s = lax.axis_index("s")
        npc = jnp.where(c == 0, CPW0, CPW1)
        base_e = jnp.where(c == 0, s * (CPW0 * CH),
                           NS * (CPW0 * CH) + s * (CPW1 * CH))

        def zi(i, _):
            for j in range(F // LANES):
                g[0][i, pl.ds(j * LANES, LANES)] = jnp.zeros((LANES,),
                                                             jnp.float32)
            return 0
        lax.fori_loop(0, CH, zi, 0)
        row0 = s * STRIPE
        for k in range(STRIPE // CH):
            pltpu.sync_copy(g[0], acc.at[pl.ds(row0 + k * CH, CH)])
        if STRIPE % CH:
            pltpu.sync_copy(g[0].at[pl.ds(0, STRIPE % CH)],
                            acc.at[pl.ds(row0 + (STRIPE // CH) * CH,
                                         STRIPE % CH)])

        @pl.when(s == 0)
        def _zrem():
            pltpu.sync_copy(g[0].at[pl.ds(0, REM)], acc.at[pl.ds(LAST0, REM)])
        plsc.subcore_barrier()

        def _scale(gb, m):
            def grp(g_, _):
                wrow = wall[m, pl.ds(g_ * LANES, LANES)]
                e0 = g_ * LANES
                for k in range(LANES):
                    wv = jnp.full((LANES,), wrow[k])
                    for j in range(FR // LANES):
                        v = gb[e0 + k, pl.ds(j * LANES, LANES)]
                        gb[e0 + k, pl.ds(j * LANES, LANES)] = v * wv
                return 0
            lax.fori_loop(0, CH // LANES, grp, 0)

        def _fire_idx(t, j):
            sl = pl.ds(base_e + t * CH, CH)
            pltpu.async_copy(src_hbm.at[sl], eall.at[2 * j], isem[j])
            pltpu.async_copy(dst_hbm.at[sl], eall.at[2 * j + 1], isem[j])
            pltpu.async_copy(w_hbm.at[sl], wall.at[j], isem[j])

        def _wait_idx(t, j):
            sl = pl.ds(base_e + t * CH, CH)
            pltpu.make_async_copy(src_hbm.at[sl], eall.at[2 * j],
                                  isem[j]).wait()
            pltpu.make_async_copy(dst_hbm.at[sl], eall.at[2 * j + 1],
                                  isem[j]).wait()
            pltpu.make_async_copy(w_hbm.at[sl], wall.at[j], isem[j]).wait()

        def _fire_gather(j6, k):
            pltpu.async_copy(y_hbm.at[eall.at[2 * j6]], g[k], gsem[k])

        def _wait_gather(j6, k):
            pltpu.make_async_copy(y_hbm.at[eall.at[2 * j6]], g[k],
                                  gsem[k]).wait()

        def _wait_scatter(j6, k):
            pltpu.make_async_copy(g[k], acc.at[eall.at[2 * j6 + 1]],
                                  ssem[k]).wait()

        sl0 = pl.ds(base_e, CH)
        pltpu.sync_copy(src_hbm.at[sl0], eall.at[0])
        pltpu.sync_copy(dst_hbm.at[sl0], eall.at[1])
        pltpu.sync_copy(w_hbm.at[sl0], wall.at[0])
        _fire_idx(1, 1)
        _fire_idx(2, 2)
        _fire_gather(0, 0)
        _wait_idx(1, 1)
        _fire_gather(1, 1)

        def ring(q, _):
            t0 = 2 * K * q
            for m in range(2 * K):
                t = t0 + m
                k = m % K

                @pl.when(t + 2 < npc)
                def _ahead(t=t, m=m, k=k):
                    @pl.when(t >= 1)
                    def _drain():
                        _wait_scatter((m - 1) % (2 * K), (k - 1) % K)
                    _wait_idx(t + 2, (m + 2) % (2 * K))
                    _fire_gather((m + 2) % (2 * K), (k + 2) % K)

                @pl.when(t + 3 < npc)
                def _pre(t=t, m=m):
                    _fire_idx(t + 3, (m + 3) % (2 * K))

                _wait_gather(m, k)
                _scale(g[k], m)
                pltpu.async_copy(g[k], acc.at[eall.at[2 * m + 1]], ssem[k],
                                 add=True)
            return 0
        lax.fori_loop(0, npc // (2 * K), ring, 0)

        for d in range(K):
            _wait_scatter(K + d, d)

        plsc.subcore_barrier()
        pltpu.sync_copy(acc.at[pl.ds(row0, STRIPE)],
                        out_hbm.at[c, pl.ds(row0, STRIPE)])

        @pl.when(s == 0)
        def _crem():
            pltpu.sync_copy(acc.at[pl.ds(LAST0, REM)],
                            out_hbm.at[c, pl.ds(LAST0, REM)])

    return segsum


_BN_SCALE = float(1.0 / (1.0 + 1e-5) ** 0.5)


def _t1_body(P_ref, x_ref, Wrel1_ref, brel1_ref, Wroot1_ref, Wrel2_ref,
             h1_ref, y2_ref):
    agg = P_ref[0] + P_ref[1]
    pre = (jnp.dot(agg, Wrel1_ref[...], preferred_element_type=jnp.float32)
           + brel1_ref[...]
           + jnp.dot(x_ref[...], Wroot1_ref[...], preferred_element_type=jnp.float32))
    h1 = jnp.maximum(pre, 0.0)
    h1_ref[...] = h1
    y2_ref[...] = jnp.dot(h1, Wrel2_ref[...], preferred_element_type=jnp.float32)


def _make_mid_body(F):
    def _mid_body(P_ref, h_ref, Wroot_ref, brel_ref, gm_ref, bm_ref, Wnext_ref,
                  hout_ref, ynext_ref):
        agg = (P_ref[0] + P_ref[1])[:, :F]
        pre = (agg + brel_ref[...]
               + jnp.dot(h_ref[...], Wroot_ref[...],
                         preferred_element_type=jnp.float32))
        hn = jnp.maximum(pre * (gm_ref[...] * _BN_SCALE) + bm_ref[...], 0.0)
        hout_ref[...] = hn
        ynext_ref[...] = jnp.dot(hn, Wnext_ref[...],
                                 preferred_element_type=jnp.float32)
    return _mid_body


def _t4_body(P_ref, h_ref, Wroot_ref, brel_ref, gm_ref, bm_ref, batch_ref,
             Wfc_ref, bfc_ref, out_ref, sums_acc, cnt_acc):
    i = pl.program_id(0)
    agg = (P_ref[0] + P_ref[1])[:, :32]
    pre = (agg + brel_ref[...]
           + jnp.dot(h_ref[...], Wroot_ref[...], preferred_element_type=jnp.float32))
    h4 = jnp.maximum(pre * (gm_ref[...] * _BN_SCALE) + bm_ref[...], 0.0)
    bb = batch_ref[0, 0, :]
    onehot = (bb[:, None] == lax.broadcasted_iota(jnp.int32, (1, G_GRAPHS), 1)
              ).astype(jnp.float32)
    ps = lax.dot_general(onehot, h4, (((0,), (0,)), ((), ())),
                         preferred_element_type=jnp.float32)
    ones = jnp.ones((BROWS, 1), jnp.float32)
    pc = lax.dot_general(onehot, ones, (((0,), (0,)), ((), ())),
                         preferred_element_type=jnp.float32)

    @pl.when(i == 0)
    def _():
        sums_acc[...] = jnp.zeros_like(sums_acc)
        cnt_acc[...] = jnp.zeros_like(cnt_acc)

    sums_acc[...] += ps
    cnt_acc[...] += pc

    @pl.when(i == pl.num_programs(0) - 1)
    def _():
        pooled = sums_acc[...] / jnp.maximum(cnt_acc[...], 1.0)
        o = jnp.dot(pooled, Wfc_ref[...], preferred_element_type=jnp.float32) \
            + bfc_ref[...]
        out_ref[...] = jax.nn.sigmoid(o)


def _full(shape):
    nd = len(shape)
    return pl.BlockSpec(shape, lambda i: (0,) * nd)


def _t1_call(P1, x, Wrel1, brel1, Wroot1, Wrel2):
    return pl.pallas_call(
        _t1_body,
        grid=(NBLK,),
        in_specs=[
            pl.BlockSpec((2, BROWS, 128), lambda i: (0, i, 0)),
            pl.BlockSpec((BROWS, 128), lambda i: (i, 0)),
            _full((128, 256)),
            _full((1, 256)),
            _full((128, 256)),
            _full((256, 128)),
        ],
        out_specs=[
            pl.BlockSpec((BROWS, 256), lambda i: (i, 0)),
            pl.BlockSpec((BROWS, 128), lambda i: (i, 0)),
        ],
        out_shape=[
            jax.ShapeDtypeStruct((N_NODES, 256), jnp.float32),
            jax.ShapeDtypeStruct((N_NODES, 128), jnp.float32),
        ],
    )(P1, x, Wrel1, brel1, Wroot1, Wrel2)


def _mid_call(P, h, Wroot, brel, gm, bm, Wnext, Fin, F):
    return pl.pallas_call(
        _make_mid_body(F),
        grid=(NBLK,),
        in_specs=[
            pl.BlockSpec((2, BROWS, 128), lambda i: (0, i, 0)),
            pl.BlockSpec((BROWS, Fin), lambda i: (i, 0)),
            _full((Fin, F)),
            _full((1, F)),
            _full((1, F)),
            _full((1, F)),
            _full((F, 128)),
        ],
        out_specs=[
            pl.BlockSpec((BROWS, F), lambda i: (i, 0)),
            pl.BlockSpec((BROWS, 128), lambda i: (i, 0)),
        ],
        out_shape=[
            jax.ShapeDtypeStruct((N_NODES, F), jnp.float32),
            jax.ShapeDtypeStruct((N_NODES, 128), jnp.float32),
        ],
    )(P, h, Wroot, brel, gm, bm, Wnext)


def _t4_call(P4, h3, Wroot4, brel4, g3, b3, batch3, Wfc, bfc):
    return pl.pallas_call(
        _t4_body,
        grid=(NBLK,),
        in_specs=[
            pl.BlockSpec((2, BROWS, 128), lambda i: (0, i, 0)),
            pl.BlockSpec((BROWS, 64), lambda i: (i, 0)),
            _full((64, 32)),
            _full((1, 32)),
            _full((1, 32)),
            _full((1, 32)),
            pl.BlockSpec((1, 1, BROWS), lambda i: (i, 0, 0)),
            _full((32, 1)),
            _full((1, 1)),
        ],
        out_specs=pl.BlockSpec((G_GRAPHS, 1), lambda i: (0, 0)),
        out_shape=jax.ShapeDtypeStruct((G_GRAPHS, 1), jnp.float32),
        scratch_shapes=[
            pltpu.VMEM((G_GRAPHS, 32), jnp.float32),
            pltpu.VMEM((G_GRAPHS, 1), jnp.float32),
        ],
        compiler_params=pltpu.CompilerParams(
            dimension_semantics=("arbitrary",)),
    )(P4, h3, Wroot4, brel4, g3, b3, batch3, Wfc, bfc)


def kernel(x, edge_index, edge_attr, batch, Wrel1, brel1, Wroot1, Wrel2, brel2,
           Wroot2, Wrel3, brel3, Wroot3, Wrel4, brel4, Wroot4, g1, b1, g2, b2,
           g3, b3, Wfc, bfc):
    pad = E_PAD - E_EDGES
    srcp = jnp.concatenate([edge_index[0], jnp.zeros((pad,), jnp.int32)])
    dstp = jnp.concatenate([edge_index[1], jnp.zeros((pad,), jnp.int32)])
    wp = jnp.concatenate([edge_attr, jnp.zeros((pad,), jnp.float32)])
    batch3 = batch.reshape(NBLK, 1, BROWS)

    brel1_2 = brel1.reshape(1, -1)
    brel2_2 = brel2.reshape(1, -1)
    brel3_2 = brel3.reshape(1, -1)
    brel4_2 = brel4.reshape(1, -1)
    g1_2, b1_2 = g1.reshape(1, -1), b1.reshape(1, -1)
    g2_2, b2_2 = g2.reshape(1, -1), b2.reshape(1, -1)
    g3_2, b3_2 = g3.reshape(1, -1), b3.reshape(1, -1)
    bfc_2 = bfc.reshape(1, 1)

    Wrel3p = jnp.pad(Wrel3, ((0, 0), (0, 64)))
    Wrel4p = jnp.pad(Wrel4, ((0, 0), (0, 96)))

    P1 = _make_segsum(128)(x, srcp, dstp, wp)
    h1, y2 = _t1_call(P1, x, Wrel1, brel1_2, Wroot1, Wrel2)
    P2 = _make_segsum(128)(y2, srcp, dstp, wp)
    h2, y3 = _mid_call(P2, h1, Wroot2, brel2_2, g1_2, b1_2, Wrel3p, 256, 128)
    P3 = _make_segsum(64)(y3, srcp, dstp, wp)
    h3, y4 = _mid_call(P3, h2, Wroot3, brel3_2, g2_2, b2_2, Wrel4p, 128, 64)
    P4 = _make_segsum(32)(y4, srcp, dstp, wp)
    out = _t4_call(P4, h3, Wroot4, brel4_2, g3_2, b3_2, batch3, Wfc, bfc_2)
    return out


# speedup vs baseline: 1.2178x; 1.2178x over previous
import functools

import jax
import jax.numpy as jnp
from jax import lax
from jax.experimental import pallas as pl
from jax.experimental.pallas import tpu as pltpu
from jax.experimental.pallas import tpu_sc as plsc

N_NODES = 10000
E_EDGES = 640000
G_GRAPHS = 64

NC, NS, LANES = 2, 16, 16
NW = NC * NS
CH = 96
K = 3
CPW0 = 276
CPW1 = 144
E_PAD = NS * (CPW0 + CPW1) * CH
F = 128

STRIPE = 624
LAST0 = NS * STRIPE
REM = N_NODES - LAST0

BROWS = 1000
NBLK = N_NODES // BROWS


@functools.lru_cache(maxsize=None)
def _make_segsum(FR):
    mesh = plsc.VectorSubcoreMesh(
        core_axis_name="c", subcore_axis_name="s", num_cores=NC, num_subcores=NS
    )

    @functools.partial(
        pl.kernel,
        out_type=jax.ShapeDtypeStruct((NC, N_NODES, F), jnp.float32),
        mesh=mesh,
        scratch_types=(
            [pltpu.VMEM_SHARED((N_NODES, F), jnp.float32)]
            + [pltpu.VMEM((4 * K, CH), jnp.int32)]
            + [pltpu.VMEM((2 * K, CH), jnp.float32)]
            + [pltpu.VMEM((CH, F), jnp.float32) for _ in range(K)]
            + [pltpu.SemaphoreType.DMA] * (4 * K)
        ),
        compiler_params=pltpu.CompilerParams(needs_layout_passes=False),
    )
    def segsum(y_hbm, src_hbm, dst_hbm, w_hbm, out_hbm, acc, eall, wall,
               *bufs):
        g = list(bufs[0:K])
        isem = list(bufs[K:3 * K])
        gsem = list(bufs[3 * K:4 * K])
        ssem = list(bufs[4 * K:5 * K])
        c = lax.axis_index("c")
        s = lax.axis_index("s")
        npc = jnp.where(c == 0, CPW0, CPW1)
        base_e = jnp.where(c == 0, s * (CPW0 * CH),
                           NS * (CPW0 * CH) + s * (CPW1 * CH))

        def zi(i, _):
            for j in range(F // LANES):
                g[0][i, pl.ds(j * LANES, LANES)] = jnp.zeros((LANES,),
                                                             jnp.float32)
            return 0
        lax.fori_loop(0, CH, zi, 0)
        row0 = s * STRIPE
        for k in range(STRIPE // CH):
            pltpu.sync_copy(g[0], acc.at[pl.ds(row0 + k * CH, CH)])
        if STRIPE % CH:
            pltpu.sync_copy(g[0].at[pl.ds(0, STRIPE % CH)],
                            acc.at[pl.ds(row0 + (STRIPE // CH) * CH,
                                         STRIPE % CH)])

        @pl.when(s == 0)
        def _zrem():
            pltpu.sync_copy(g[0].at[pl.ds(0, REM)], acc.at[pl.ds(LAST0, REM)])
        plsc.subcore_barrier()

        def _scale(gb, m):
            def grp(g_, _):
                wrow = wall[m, pl.ds(g_ * LANES, LANES)]
                e0 = g_ * LANES
                for k in range(LANES):
                    wv = jnp.full((LANES,), wrow[k])
                    for j in range(FR // LANES):
                        v = gb[e0 + k, pl.ds(j * LANES, LANES)]
                        gb[e0 + k, pl.ds(j * LANES, LANES)] = v * wv
                return 0
            lax.fori_loop(0, CH // LANES, grp, 0)

        def _fire_idx(t, j):
            sl = pl.ds(base_e + t * CH, CH)
            pltpu.async_copy(src_hbm.at[sl], eall.at[2 * j], isem[j])
            pltpu.async_copy(dst_hbm.at[sl], eall.at[2 * j + 1], isem[j])
            pltpu.async_copy(w_hbm.at[sl], wall.at[j], isem[j])

        def _wait_idx(t, j):
            sl = pl.ds(base_e + t * CH, CH)
            pltpu.make_async_copy(src_hbm.at[sl], eall.at[2 * j],
                                  isem[j]).wait()
            pltpu.make_async_copy(dst_hbm.at[sl], eall.at[2 * j + 1],
                                  isem[j]).wait()
            pltpu.make_async_copy(w_hbm.at[sl], wall.at[j], isem[j]).wait()

        def _fire_gather(j6, k):
            pltpu.async_copy(y_hbm.at[eall.at[2 * j6]], g[k], gsem[k])

        def _wait_gather(j6, k):
            pltpu.make_async_copy(y_hbm.at[eall.at[2 * j6]], g[k],
                                  gsem[k]).wait()

        def _wait_scatter(j6, k):
            pltpu.make_async_copy(g[k], acc.at[eall.at[2 * j6 + 1]],
                                  ssem[k]).wait()

        sl0 = pl.ds(base_e, CH)
        pltpu.sync_copy(src_hbm.at[sl0], eall.at[0])
        pltpu.sync_copy(dst_hbm.at[sl0], eall.at[1])
        pltpu.sync_copy(w_hbm.at[sl0], wall.at[0])
        _fire_idx(1, 1)
        _fire_idx(2, 2)
        _fire_gather(0, 0)
        _wait_idx(1, 1)
        _fire_gather(1, 1)

        def ring(q, _):
            t0 = 2 * K * q
            for m in range(2 * K):
                t = t0 + m
                k = m % K

                @pl.when(t + 2 < npc)
                def _ahead(t=t, m=m, k=k):
                    @pl.when(t >= 1)
                    def _drain():
                        _wait_scatter((m - 1) % (2 * K), (k - 1) % K)
                    _wait_idx(t + 2, (m + 2) % (2 * K))
                    _fire_gather((m + 2) % (2 * K), (k + 2) % K)

                @pl.when(t + 3 < npc)
                def _pre(t=t, m=m):
                    _fire_idx(t + 3, (m + 3) % (2 * K))

                _wait_gather(m, k)
                _scale(g[k], m)
                pltpu.async_copy(g[k], acc.at[eall.at[2 * m + 1]], ssem[k],
                                 add=True)
            return 0
        lax.fori_loop(0, npc // (2 * K), ring, 0)

        for d in range(K):
            _wait_scatter(K + d, d)

        plsc.subcore_barrier()
        pltpu.sync_copy(acc.at[pl.ds(row0, STRIPE)],
                        out_hbm.at[c, pl.ds(row0, STRIPE)])

        @pl.when(s == 0)
        def _crem():
            pltpu.sync_copy(acc.at[pl.ds(LAST0, REM)],
                            out_hbm.at[c, pl.ds(LAST0, REM)])

    return segsum


_BN_SCALE = float(1.0 / (1.0 + 1e-5) ** 0.5)


def _t1_body(P_ref, x_ref, Wrel1_ref, brel1_ref, Wroot1_ref, Wrel2_ref,
             h1_ref, y2_ref):
    agg = P_ref[0] + P_ref[1]
    pre = (jnp.dot(agg, Wrel1_ref[...], preferred_element_type=jnp.float32)
           + brel1_ref[...]
           + jnp.dot(x_ref[...], Wroot1_ref[...], preferred_element_type=jnp.float32))
    h1 = jnp.maximum(pre, 0.0)
    h1_ref[...] = h1
    y2_ref[...] = jnp.dot(h1, Wrel2_ref[...], preferred_element_type=jnp.float32)


def _make_mid_body(F):
    def _mid_body(P_ref, h_ref, Wroot_ref, brel_ref, gm_ref, bm_ref, Wnext_ref,
                  hout_ref, ynext_ref):
        agg = (P_ref[0] + P_ref[1])[:, :F]
        pre = (agg + brel_ref[...]
               + jnp.dot(h_ref[...], Wroot_ref[...],
                         preferred_element_type=jnp.float32))
        hn = jnp.maximum(pre * (gm_ref[...] * _BN_SCALE) + bm_ref[...], 0.0)
        hout_ref[...] = hn
        ynext_ref[...] = jnp.dot(hn, Wnext_ref[...],
                                 preferred_element_type=jnp.float32)
    return _mid_body


def _t4_body(P_ref, h_ref, Wroot_ref, brel_ref, gm_ref, bm_ref, batch_ref,
             Wfc_ref, bfc_ref, out_ref, sums_acc, cnt_acc):
    i = pl.program_id(0)
    agg = (P_ref[0] + P_ref[1])[:, :32]
    pre = (agg + brel_ref[...]
           + jnp.dot(h_ref[...], Wroot_ref[...], preferred_element_type=jnp.float32))
    h4 = jnp.maximum(pre * (gm_ref[...] * _BN_SCALE) + bm_ref[...], 0.0)
    bb = batch_ref[0, 0, :]
    onehot = (bb[:, None] == lax.broadcasted_iota(jnp.int32, (1, G_GRAPHS), 1)
              ).astype(jnp.float32)
    ps = lax.dot_general(onehot, h4, (((0,), (0,)), ((), ())),
                         preferred_element_type=jnp.float32)
    ones = jnp.ones((BROWS, 1), jnp.float32)
    pc = lax.dot_general(onehot, ones, (((0,), (0,)), ((), ())),
                         preferred_element_type=jnp.float32)

    @pl.when(i == 0)
    def _():
        sums_acc[...] = jnp.zeros_like(sums_acc)
        cnt_acc[...] = jnp.zeros_like(cnt_acc)

    sums_acc[...] += ps
    cnt_acc[...] += pc

    @pl.when(i == pl.num_programs(0) - 1)
    def _():
        pooled = sums_acc[...] / jnp.maximum(cnt_acc[...], 1.0)
        o = jnp.dot(pooled, Wfc_ref[...], preferred_element_type=jnp.float32) \
            + bfc_ref[...]
        out_ref[...] = jax.nn.sigmoid(o)


def _full(shape):
    nd = len(shape)
    return pl.BlockSpec(shape, lambda i: (0,) * nd)


def _t1_call(P1, x, Wrel1, brel1, Wroot1, Wrel2):
    return pl.pallas_call(
        _t1_body,
        grid=(NBLK,),
        in_specs=[
            pl.BlockSpec((2, BROWS, 128), lambda i: (0, i, 0)),
            pl.BlockSpec((BROWS, 128), lambda i: (i, 0)),
            _full((128, 256)),
            _full((1, 256)),
            _full((128, 256)),
            _full((256, 128)),
        ],
        out_specs=[
            pl.BlockSpec((BROWS, 256), lambda i: (i, 0)),
            pl.BlockSpec((BROWS, 128), lambda i: (i, 0)),
        ],
        out_shape=[
            jax.ShapeDtypeStruct((N_NODES, 256), jnp.float32),
            jax.ShapeDtypeStruct((N_NODES, 128), jnp.float32),
        ],
    )(P1, x, Wrel1, brel1, Wroot1, Wrel2)


def _mid_call(P, h, Wroot, brel, gm, bm, Wnext, Fin, F):
    return pl.pallas_call(
        _make_mid_body(F),
        grid=(NBLK,),
        in_specs=[
            pl.BlockSpec((2, BROWS, 128), lambda i: (0, i, 0)),
            pl.BlockSpec((BROWS, Fin), lambda i: (i, 0)),
            _full((Fin, F)),
            _full((1, F)),
            _full((1, F)),
            _full((1, F)),
            _full((F, 128)),
        ],
        out_specs=[
            pl.BlockSpec((BROWS, F), lambda i: (i, 0)),
            pl.BlockSpec((BROWS, 128), lambda i: (i, 0)),
        ],
        out_shape=[
            jax.ShapeDtypeStruct((N_NODES, F), jnp.float32),
            jax.ShapeDtypeStruct((N_NODES, 128), jnp.float32),
        ],
    )(P, h, Wroot, brel, gm, bm, Wnext)


def _t4_call(P4, h3, Wroot4, brel4, g3, b3, batch3, Wfc, bfc):
    return pl.pallas_call(
        _t4_body,
        grid=(NBLK,),
        in_specs=[
            pl.BlockSpec((2, BROWS, 128), lambda i: (0, i, 0)),
            pl.BlockSpec((BROWS, 64), lambda i: (i, 0)),
            _full((64, 32)),
            _full((1, 32)),
            _full((1, 32)),
            _full((1, 32)),
            pl.BlockSpec((1, 1, BROWS), lambda i: (i, 0, 0)),
            _full((32, 1)),
            _full((1, 1)),
        ],
        out_specs=pl.BlockSpec((G_GRAPHS, 1), lambda i: (0, 0)),
        out_shape=jax.ShapeDtypeStruct((G_GRAPHS, 1), jnp.float32),
        scratch_shapes=[
            pltpu.VMEM((G_GRAPHS, 32), jnp.float32),
            pltpu.VMEM((G_GRAPHS, 1), jnp.float32),
        ],
        compiler_params=pltpu.CompilerParams(
            dimension_semantics=("arbitrary",)),
    )(P4, h3, Wroot4, brel4, g3, b3, batch3, Wfc, bfc)


def kernel(x, edge_index, edge_attr, batch, Wrel1, brel1, Wroot1, Wrel2, brel2,
           Wroot2, Wrel3, brel3, Wroot3, Wrel4, brel4, Wroot4, g1, b1, g2, b2,
           g3, b3, Wfc, bfc):
    pad = E_PAD - E_EDGES
    srcp = jnp.concatenate([edge_index[0], jnp.zeros((pad,), jnp.int32)])
    dstp = jnp.concatenate([edge_index[1], jnp.zeros((pad,), jnp.int32)])
    wp = jnp.concatenate([edge_attr, jnp.zeros((pad,), jnp.float32)])
    batch3 = batch.reshape(NBLK, 1, BROWS)

    brel1_2 = brel1.reshape(1, -1)
    brel2_2 = brel2.reshape(1, -1)
    brel3_2 = brel3.reshape(1, -1)
    brel4_2 = brel4.reshape(1, -1)
    g1_2, b1_2 = g1.reshape(1, -1), b1.reshape(1, -1)
    g2_2, b2_2 = g2.reshape(1, -1), b2.reshape(1, -1)
    g3_2, b3_2 = g3.reshape(1, -1), b3.reshape(1, -1)
    bfc_2 = bfc.reshape(1, 1)

    Wrel3p = jnp.pad(Wrel3, ((0, 0), (0, 64)))
    Wrel4p = jnp.pad(Wrel4, ((0, 0), (0, 96)))

    P1 = _make_segsum(128)(x, srcp, dstp, wp)
    h1, y2 = _t1_call(P1, x, Wrel1, brel1_2, Wroot1, Wrel2)
    P2 = _make_segsum(128)(y2, srcp, dstp, wp)
    h2, y3 = _mid_call(P2, h1, Wroot2, brel2_2, g1_2, b1_2, Wrel3p, 256, 128)
    P3 = _make_segsum(64)(y3, srcp, dstp, wp)
    h3, y4 = _mid_call(P3, h2, Wroot3, brel3_2, g2_2, b2_2, Wrel4p, 128, 64)
    P4 = _make_segsum(32)(y4, srcp, dstp, wp)
    out = _t4_call(P4, h3, Wroot4, brel4_2, g3_2, b3_2, batch3, Wfc, bfc_2)
    return out
